# baseline (speedup 1.0000x reference)
import jax
import jax.numpy as jnp
from jax.experimental import pallas as pl
from jax.experimental.pallas import tpu as pltpu

F32 = jnp.float32
P = 7
WIN = 37
STEP = 4
R = WIN // 2
M1, M2 = 18, 55
H = W = 224
C = 3
HC = H - P + 1
L = HC * HC
NA = 221
G = 56
B = G * G
NOFF = WIN * WIN
CENTER = R * WIN + R
NF1 = P * P
NF = C * NF1
NFA = NF + 1


def _unfold(img, c):
    cols = []
    for ch in range(c):
        for i in range(P):
            for j in range(P):
                cols.append(jax.lax.slice(img, (ch, i, j), (ch + 1, i + HC, j + HC))[0])
    return jnp.stack(cols, axis=0)


def _patch_table(img):
    cols = []
    for ch in range(C):
        for i in range(P):
            for j in range(P):
                cols.append(jax.lax.slice(img, (ch, i, j), (ch + 1, i + HC, j + HC))[0])
    return jnp.stack(cols, axis=-1).reshape(L, NF)


def _align_inf(x):
    inf = jnp.float32(jnp.inf)
    xp = jnp.pad(x, ((0, 0), (0, 3), (0, 3)), constant_values=inf)
    xp = xp.at[:, -1:, :HC:STEP].set(x[:, -1:, ::STEP])
    xp = xp.at[:, :HC:STEP, -1:].set(x[:, ::STEP, -1:])
    xp = xp.at[:, -1:, -1:].set(x[:, -1:, -1:])
    xp = xp.at[:, HC - 1:HC, :HC:STEP].set(inf)
    xp = xp.at[:, :HC:STEP, HC - 1:HC].set(inf)
    xp = xp.at[:, HC - 1:HC, HC - 1:HC].set(inf)
    return xp


def _phases(xp):
    inf = jnp.float32(jnp.inf)
    big = jnp.pad(xp, ((0, 0), (R, 260 - NA - R), (R, 260 - NA - R)),
                  constant_values=inf)
    r = big.reshape(NF1, 65, STEP, 65, STEP)
    r = jnp.transpose(r, (2, 4, 0, 1, 3))
    return r.reshape(16, NF1, 65, 65)


def _dist_kernel(xph_ref, xc_ref, o_ref):
    i = pl.program_id(0)
    pi = i % STEP
    a0 = i // STEP
    xc = xc_ref[...]
    for j in range(WIN):
        pj = j % STEP
        b0 = j // STEP
        sl = xph_ref[pi * STEP + pj, :, pl.ds(a0, G), b0:b0 + G]
        d = sl - xc
        o_ref[0, j] = jnp.sum(d * d, axis=0)


def _distances(xph, xc):
    out = pl.pallas_call(
        _dist_kernel,
        grid=(WIN,),
        in_specs=[
            pl.BlockSpec((16, NF1, 65, 65), lambda i: (0, 0, 0, 0)),
            pl.BlockSpec((NF1, G, G), lambda i: (0, 0, 0)),
        ],
        out_specs=pl.BlockSpec((1, WIN, G, G), lambda i: (i, 0, 0, 0)),
        out_shape=jax.ShapeDtypeStruct((WIN, WIN, G, G), F32),
    )(xph, xc)
    return out.reshape(NOFF, B).T


def _topk_kernel(d_ref, o_ref, *, m, rb):
    d = d_ref[...]
    lane = jax.lax.broadcasted_iota(jnp.int32, (rb, NOFF), 1)
    d = jnp.where(lane == CENTER, -jnp.inf, d)

    lanem = jax.lax.broadcasted_iota(jnp.int32, (rb, m), 1)

    def body(k, carry):
        dcur, out = carry
        dmin = jnp.min(dcur, axis=1, keepdims=True)
        idx = jnp.min(jnp.where(dcur == dmin, lane, NOFF), axis=1)
        out = jnp.where(lanem == k, idx[:, None], out)
        dcur = jnp.where(lane == idx[:, None], jnp.inf, dcur)
        return dcur, out

    _, idx = jax.lax.fori_loop(
        0, m, body, (d, jnp.zeros((rb, m), jnp.int32)))
    g = pl.program_id(0) * rb + jax.lax.broadcasted_iota(jnp.int32, (rb, m), 0)
    cr = jnp.minimum(STEP * (g // G), HC - 1)
    cc = jnp.minimum(STEP * (g % G), HC - 1)
    ir = jnp.minimum(idx // WIN - R + cr, HC - 1)
    ic = jnp.minimum(idx % WIN - R + cc, HC - 1)
    o_ref[...] = ir * HC + ic


def _topk_indices(dist, m):
    rb = 392
    import functools
    return pl.pallas_call(
        functools.partial(_topk_kernel, m=m, rb=rb),
        grid=(B // rb,),
        in_specs=[pl.BlockSpec((rb, NOFF), lambda b: (b, 0))],
        out_specs=pl.BlockSpec((rb, m), lambda b: (b, 0)),
        out_shape=jax.ShapeDtypeStruct((B, m), jnp.int32),
    )(dist)


def _gather_kernel(idx_ref, tab_ref, o_ref, *, m):
    def body(k, carry):
        for q in range(8):
            t = idx_ref[q, k]
            o_ref[q, pl.ds(k, 1), :] = tab_ref[pl.ds(t, 1), :]
        return carry

    jax.lax.fori_loop(0, m, body, 0)


def _gather(tab, idx, m):
    import functools
    return pl.pallas_call(
        functools.partial(_gather_kernel, m=m),
        grid=(B // 8,),
        in_specs=[
            pl.BlockSpec((8, m), lambda b: (b, 0), memory_space=pltpu.SMEM),
            pl.BlockSpec((L, NF), lambda b: (0, 0)),
        ],
        out_specs=pl.BlockSpec((8, m, NF), lambda b: (b, 0, 0)),
        out_shape=jax.ShapeDtypeStruct((B, m, NF), F32),
    )(idx, tab)


def _solve_kernel(ns2_ref, y_ref, s_ref, xw_ref, *, m, gb, ridge):
    ns2 = ns2_ref[0, 0]
    Y = y_ref[...]
    S = s_ref[...]
    Gm = jax.lax.dot_general(S, S, (((2,), (2,)), ((0,), (0,))),
                             preferred_element_type=F32)
    ii = jax.lax.broadcasted_iota(jnp.int32, (1, m, m), 1)
    jj = jax.lax.broadcasted_iota(jnp.int32, (1, m, m), 2)
    eye = jnp.where(ii == jj, ns2, jnp.float32(0.0))
    if ridge:
        A = Gm + eye
        Bm = Gm
    else:
        A = Gm
        Bm = Gm - eye
    M = jnp.concatenate([A, Bm], axis=2)
    lane = jax.lax.broadcasted_iota(jnp.int32, (gb, m, 2 * m), 2)
    subl = jax.lax.broadcasted_iota(jnp.int32, (gb, m, 2 * m), 1)

    def body(k, Mc):
        row = jnp.sum(jnp.where(subl == k, Mc, 0.0),
                      axis=1, keepdims=True)
        piv = jnp.sum(jnp.where(lane[:, :1, :] == k, row, 0.0),
                      axis=2, keepdims=True)
        row = row / piv
        colk = jnp.sum(jnp.where(lane == k, Mc, 0.0),
                       axis=2, keepdims=True)
        Mn = Mc - colk * row
        return jnp.where(subl == k, row, Mn)

    M = jax.lax.fori_loop(0, m, body, M)
    X = M[:, :, m:]
    Xh = jax.lax.dot_general(X, Y, (((1,), (1,)), ((0,), (0,))),
                             preferred_element_type=F32)
    w = 1.0 / jnp.sum(X * X, axis=1)
    xw_ref[:, :, :NF] = Xh * w[:, :, None]
    xw_ref[:, :, NF:] = w[:, :, None]


def _solve(Yv, Sv, ns2, m, ridge):
    import functools
    gb = 16 if m == M2 else 32
    nblk = B // gb
    return pl.pallas_call(
        functools.partial(_solve_kernel, m=m, gb=gb, ridge=ridge),
        grid=(nblk,),
        in_specs=[
            pl.BlockSpec(memory_space=pltpu.SMEM),
            pl.BlockSpec((gb, m, NF), lambda b: (b, 0, 0)),
            pl.BlockSpec((gb, m, NF), lambda b: (b, 0, 0)),
        ],
        out_specs=pl.BlockSpec((gb, m, NFA), lambda b: (b, 0, 0)),
        out_shape=jax.ShapeDtypeStruct((B, m, NFA), F32),
    )(ns2, Yv, Sv)


def _scatter_kernel(idx_ref, v_ref, acc_ref, *, m):
    @pl.when(pl.program_id(0) == 0)
    def _():
        acc_ref[...] = jnp.zeros_like(acc_ref)

    def body(k, carry):
        for q in range(8):
            t = idx_ref[q, k]
            acc_ref[pl.ds(t, 1), :] += v_ref[q, pl.ds(k, 1), :]
        return carry

    jax.lax.fori_loop(0, m, body, 0)


def _scatter(idx, vals, m):
    import functools
    return pl.pallas_call(
        functools.partial(_scatter_kernel, m=m),
        grid=(B // 8,),
        in_specs=[
            pl.BlockSpec((8, m), lambda b: (b, 0), memory_space=pltpu.SMEM),
            pl.BlockSpec((8, m, NFA), lambda b: (b, 0, 0)),
        ],
        out_specs=pl.BlockSpec((L, NFA), lambda b: (0, 0)),
        out_shape=jax.ShapeDtypeStruct((L, NFA), F32),
    )(idx, vals)


def _fold_kernel(accT_ref, o_ref, g_ref, den_ref):
    o_ref[...] = jnp.zeros_like(o_ref)
    den_ref[...] = jnp.zeros_like(den_ref)
    for i in range(P):
        for j in range(P):
            den_ref[i:i + HC, j:j + HC] += accT_ref[NF]
            for c in range(C):
                o_ref[c, i:i + HC, j:j + HC] += accT_ref[c * NF1 + i * P + j]
    out = o_ref[...] / den_ref[...][None]
    o_ref[...] = out
    g_ref[...] = (out[0] + out[1] + out[2]) * jnp.float32(1.0 / 3.0)


def _fold_divide(acc):
    accT = acc.T.reshape(NFA, HC, HC)
    return pl.pallas_call(
        _fold_kernel,
        out_shape=[
            jax.ShapeDtypeStruct((C, H, W), F32),
            jax.ShapeDtypeStruct((H, W), F32),
        ],
        scratch_shapes=[pltpu.VMEM((H, W), F32)],
    )(accT)


def _mean_kernel(y_ref, o_ref):
    o_ref[...] = (y_ref[0] + y_ref[1] + y_ref[2]) * jnp.float32(1.0 / 3.0)


def _chan_mean(y):
    return pl.pallas_call(
        _mean_kernel,
        out_shape=jax.ShapeDtypeStruct((H, W), F32),
    )(y)


def _one_step(guide, y_tab, s_tab, ns2, m, ridge):
    pat = _unfold(guide[None], 1)
    xp = _align_inf(pat)
    xc = xp[:, ::STEP, ::STEP]
    xph = _phases(xp)
    dist = _distances(xph, xc)
    idx = _topk_indices(dist, m)
    Yv = _gather(y_tab, idx, m)
    Sv = Yv if s_tab is None else _gather(s_tab, idx, m)
    vals = _solve(Yv, Sv, ns2, m, ridge)
    acc = _scatter(idx, vals, m)
    return _fold_divide(acc)


def kernel(input_y, sigma):
    y = input_y[0].astype(F32)
    sig = jnp.asarray(sigma, F32)
    ns2 = (jnp.float32(NF) * sig * sig).reshape(1, 1)
    y_tab = _patch_table(y)
    g1 = _chan_mean(y)
    x1, g2 = _one_step(g1, y_tab, None, ns2, M1, ridge=False)
    x1_tab = _patch_table(x1)
    x2, _ = _one_step(g2, y_tab, x1_tab, ns2, M2, ridge=True)
    return x2[None]


# speedup vs baseline: 1.7902x; 1.7902x over previous
import jax
import jax.numpy as jnp
from jax.experimental import pallas as pl
from jax.experimental.pallas import tpu as pltpu

F32 = jnp.float32
P = 7
WIN = 37
STEP = 4
R = WIN // 2
M1, M2 = 18, 55
H = W = 224
C = 3
HC = H - P + 1
L = HC * HC
NA = 221
G = 56
B = G * G
NOFF = WIN * WIN
CENTER = R * WIN + R
NF1 = P * P
NF = C * NF1
NFA = NF + 1


def _unfold(img, c):
    cols = []
    for ch in range(c):
        for i in range(P):
            for j in range(P):
                cols.append(jax.lax.slice(img, (ch, i, j), (ch + 1, i + HC, j + HC))[0])
    return jnp.stack(cols, axis=0)


def _patch_table(img):
    return _unfold(img, C).reshape(NF, L).T


def _align_inf(x):
    inf = jnp.float32(jnp.inf)
    xp = jnp.pad(x, ((0, 0), (0, 3), (0, 3)), constant_values=inf)
    xp = xp.at[:, -1:, :HC:STEP].set(x[:, -1:, ::STEP])
    xp = xp.at[:, :HC:STEP, -1:].set(x[:, ::STEP, -1:])
    xp = xp.at[:, -1:, -1:].set(x[:, -1:, -1:])
    xp = xp.at[:, HC - 1:HC, :HC:STEP].set(inf)
    xp = xp.at[:, :HC:STEP, HC - 1:HC].set(inf)
    xp = xp.at[:, HC - 1:HC, HC - 1:HC].set(inf)
    return xp


def _phases(xp):
    inf = jnp.float32(jnp.inf)
    big = jnp.pad(xp, ((0, 0), (R, 260 - NA - R), (R, 260 - NA - R)),
                  constant_values=inf)
    r = big.reshape(NF1, 65, STEP, 65, STEP)
    r = jnp.transpose(r, (2, 4, 0, 1, 3))
    return r.reshape(16, NF1, 65, 65)


def _dist_kernel(xph_ref, xc_ref, o_ref):
    i = pl.program_id(0)
    pi = i % STEP
    a0 = i // STEP
    xc = xc_ref[...]
    for j in range(WIN):
        pj = j % STEP
        b0 = j // STEP
        sl = xph_ref[pi * STEP + pj, :, pl.ds(a0, G), b0:b0 + G]
        d = sl - xc
        o_ref[0, j] = jnp.sum(d * d, axis=0)


def _distances(xph, xc):
    out = pl.pallas_call(
        _dist_kernel,
        grid=(WIN,),
        in_specs=[
            pl.BlockSpec((16, NF1, 65, 65), lambda i: (0, 0, 0, 0)),
            pl.BlockSpec((NF1, G, G), lambda i: (0, 0, 0)),
        ],
        out_specs=pl.BlockSpec((1, WIN, G, G), lambda i: (i, 0, 0, 0)),
        out_shape=jax.ShapeDtypeStruct((WIN, WIN, G, G), F32),
    )(xph, xc)
    return out.reshape(NOFF, B).T


def _topk_kernel(d_ref, o_ref, *, m, rb):
    d = d_ref[...]
    lane = jax.lax.broadcasted_iota(jnp.int32, (rb, NOFF), 1)
    d = jnp.where(lane == CENTER, -jnp.inf, d)

    lanem = jax.lax.broadcasted_iota(jnp.int32, (rb, m), 1)

    def body(k, carry):
        dcur, out = carry
        dmin = jnp.min(dcur, axis=1, keepdims=True)
        idx = jnp.min(jnp.where(dcur == dmin, lane, NOFF), axis=1)
        out = jnp.where(lanem == k, idx[:, None], out)
        dcur = jnp.where(lane == idx[:, None], jnp.inf, dcur)
        return dcur, out

    _, idx = jax.lax.fori_loop(
        0, m, body, (d, jnp.zeros((rb, m), jnp.int32)))
    g = pl.program_id(0) * rb + jax.lax.broadcasted_iota(jnp.int32, (rb, m), 0)
    cr = jnp.minimum(STEP * (g // G), HC - 1)
    cc = jnp.minimum(STEP * (g % G), HC - 1)
    ir = jnp.minimum(idx // WIN - R + cr, HC - 1)
    ic = jnp.minimum(idx % WIN - R + cc, HC - 1)
    o_ref[...] = ir * HC + ic


def _topk_indices(dist, m):
    rb = 392
    import functools
    return pl.pallas_call(
        functools.partial(_topk_kernel, m=m, rb=rb),
        grid=(B // rb,),
        in_specs=[pl.BlockSpec((rb, NOFF), lambda b: (b, 0))],
        out_specs=pl.BlockSpec((rb, m), lambda b: (b, 0)),
        out_shape=jax.ShapeDtypeStruct((B, m), jnp.int32),
    )(dist)


def _gather_kernel(idx_ref, tab_ref, o_ref, *, m):
    def body(k, carry):
        for q in range(8):
            t = idx_ref[q, k]
            o_ref[q, pl.ds(k, 1), :] = tab_ref[pl.ds(t, 1), :]
        return carry

    jax.lax.fori_loop(0, m, body, 0)


def _gather(tab, idx, m):
    import functools
    return pl.pallas_call(
        functools.partial(_gather_kernel, m=m),
        grid=(B // 8,),
        in_specs=[
            pl.BlockSpec((8, m), lambda b: (b, 0), memory_space=pltpu.SMEM),
            pl.BlockSpec((L, NF), lambda b: (0, 0)),
        ],
        out_specs=pl.BlockSpec((8, m, NF), lambda b: (b, 0, 0)),
        out_shape=jax.ShapeDtypeStruct((B, m, NF), F32),
    )(idx, tab)


def _solve_kernel(ns2_ref, y_ref, s_ref, xw_ref, *, m, gb, ridge):
    ns2 = ns2_ref[0, 0]
    Y = y_ref[...]
    S = s_ref[...]
    Gm = jax.lax.dot_general(S, S, (((2,), (2,)), ((0,), (0,))),
                             preferred_element_type=F32)
    ii = jax.lax.broadcasted_iota(jnp.int32, (1, m, m), 1)
    jj = jax.lax.broadcasted_iota(jnp.int32, (1, m, m), 2)
    eye = jnp.where(ii == jj, ns2, jnp.float32(0.0))
    if ridge:
        A = Gm + eye
        Bm = Gm
    else:
        A = Gm
        Bm = Gm - eye
    M = jnp.concatenate([A, Bm], axis=2)
    lane = jax.lax.broadcasted_iota(jnp.int32, (gb, m, 2 * m), 2)
    subl = jax.lax.broadcasted_iota(jnp.int32, (gb, m, 2 * m), 1)

    def body(k, Mc):
        row = jnp.sum(jnp.where(subl == k, Mc, 0.0),
                      axis=1, keepdims=True)
        piv = jnp.sum(jnp.where(lane[:, :1, :] == k, row, 0.0),
                      axis=2, keepdims=True)
        row = row / piv
        colk = jnp.sum(jnp.where(lane == k, Mc, 0.0),
                       axis=2, keepdims=True)
        Mn = Mc - colk * row
        return jnp.where(subl == k, row, Mn)

    M = jax.lax.fori_loop(0, m, body, M)
    X = M[:, :, m:]
    Xh = jax.lax.dot_general(X, Y, (((1,), (1,)), ((0,), (0,))),
                             preferred_element_type=F32)
    w = 1.0 / jnp.sum(X * X, axis=1)
    xw_ref[:, :, :NF] = Xh * w[:, :, None]
    xw_ref[:, :, NF:] = w[:, :, None]


def _solve(Yv, Sv, ns2, m, ridge):
    import functools
    gb = 16 if m == M2 else 32
    nblk = B // gb
    return pl.pallas_call(
        functools.partial(_solve_kernel, m=m, gb=gb, ridge=ridge),
        grid=(nblk,),
        in_specs=[
            pl.BlockSpec(memory_space=pltpu.SMEM),
            pl.BlockSpec((gb, m, NF), lambda b: (b, 0, 0)),
            pl.BlockSpec((gb, m, NF), lambda b: (b, 0, 0)),
        ],
        out_specs=pl.BlockSpec((gb, m, NFA), lambda b: (b, 0, 0)),
        out_shape=jax.ShapeDtypeStruct((B, m, NFA), F32),
    )(ns2, Yv, Sv)


def _scatter_kernel(idx_ref, v_ref, acc_ref, *, m):
    @pl.when(pl.program_id(0) == 0)
    def _():
        acc_ref[...] = jnp.zeros_like(acc_ref)

    def body(k, carry):
        for q in range(8):
            t = idx_ref[q, k]
            acc_ref[pl.ds(t, 1), :] += v_ref[q, pl.ds(k, 1), :]
        return carry

    jax.lax.fori_loop(0, m, body, 0)


def _scatter(idx, vals, m):
    import functools
    return pl.pallas_call(
        functools.partial(_scatter_kernel, m=m),
        grid=(B // 8,),
        in_specs=[
            pl.BlockSpec((8, m), lambda b: (b, 0), memory_space=pltpu.SMEM),
            pl.BlockSpec((8, m, NFA), lambda b: (b, 0, 0)),
        ],
        out_specs=pl.BlockSpec((L, NFA), lambda b: (0, 0)),
        out_shape=jax.ShapeDtypeStruct((L, NFA), F32),
    )(idx, vals)


def _fold_kernel(accT_ref, o_ref, g_ref, den_ref):
    o_ref[...] = jnp.zeros_like(o_ref)
    den_ref[...] = jnp.zeros_like(den_ref)
    for i in range(P):
        for j in range(P):
            den_ref[i:i + HC, j:j + HC] += accT_ref[NF]
            for c in range(C):
                o_ref[c, i:i + HC, j:j + HC] += accT_ref[c * NF1 + i * P + j]
    out = o_ref[...] / den_ref[...][None]
    o_ref[...] = out
    g_ref[...] = (out[0] + out[1] + out[2]) * jnp.float32(1.0 / 3.0)


def _fold_divide(acc):
    accT = acc.T.reshape(NFA, HC, HC)
    return pl.pallas_call(
        _fold_kernel,
        out_shape=[
            jax.ShapeDtypeStruct((C, H, W), F32),
            jax.ShapeDtypeStruct((H, W), F32),
        ],
        scratch_shapes=[pltpu.VMEM((H, W), F32)],
    )(accT)


def _mean_kernel(y_ref, o_ref):
    o_ref[...] = (y_ref[0] + y_ref[1] + y_ref[2]) * jnp.float32(1.0 / 3.0)


def _chan_mean(y):
    return pl.pallas_call(
        _mean_kernel,
        out_shape=jax.ShapeDtypeStruct((H, W), F32),
    )(y)


def _one_step(guide, y_tab, s_tab, ns2, m, ridge):
    pat = _unfold(guide[None], 1)
    xp = _align_inf(pat)
    xc = xp[:, ::STEP, ::STEP]
    xph = _phases(xp)
    dist = _distances(xph, xc)
    idx = _topk_indices(dist, m)
    Yv = _gather(y_tab, idx, m)
    Sv = Yv if s_tab is None else _gather(s_tab, idx, m)
    vals = _solve(Yv, Sv, ns2, m, ridge)
    acc = _scatter(idx, vals, m)
    return _fold_divide(acc)


def kernel(input_y, sigma):
    y = input_y[0].astype(F32)
    sig = jnp.asarray(sigma, F32)
    ns2 = (jnp.float32(NF) * sig * sig).reshape(1, 1)
    y_tab = _patch_table(y)
    g1 = _chan_mean(y)
    x1, g2 = _one_step(g1, y_tab, None, ns2, M1, ridge=False)
    x1_tab = _patch_table(x1)
    x2, _ = _one_step(g2, y_tab, x1_tab, ns2, M2, ridge=True)
    return x2[None]
